# baseline (speedup 1.0000x reference)
import functools

import jax
import jax.numpy as jnp
from jax import lax
from jax.experimental import pallas as pl
from jax.experimental.pallas import tpu as pltpu
from jax.experimental.pallas import tpu_sc as plsc

N = 10000
E = 320000
D_IN = 128
HIDDEN = 128
CLASSES = 64

NC = 2
NS = 16
NW = NC * NS
CHUNK = 128
NCHUNK = 80
E_PAD = NW * NCHUNK * CHUNK
N_PAD = 10240
RPT = N_PAD // NS
TCB = 1024
DEGW = 16


_SC_PARAMS = pltpu.CompilerParams(use_tc_tiling_on_sc=False)


def _sc_mesh():
  return plsc.VectorSubcoreMesh(
      core_axis_name="c", subcore_axis_name="s", num_cores=NC,
      num_subcores=NS)


def _deg_body(dsts_hbm, ones_hbm, zeros_hbm, out_hbm, acc, dsts_v, ones_v,
              zbuf_v):
  c = lax.axis_index("c")
  s = lax.axis_index("s")
  wid = s * NC + c
  pltpu.sync_copy(zeros_hbm, zbuf_v)
  pltpu.sync_copy(ones_hbm, ones_v)
  pltpu.sync_copy(dsts_hbm.at[wid], dsts_v)
  for k in range(RPT // CHUNK):
    pltpu.sync_copy(zbuf_v, acc.at[pl.ds(s * RPT + k * CHUNK, CHUNK)])
  plsc.subcore_barrier()

  def body(j, carry):
    pltpu.sync_copy(ones_v, acc.at[dsts_v.at[j]], add=True)
    return carry

  lax.fori_loop(0, NCHUNK, body, 0)
  plsc.subcore_barrier()
  for k in range(RPT // CHUNK):
    pltpu.sync_copy(acc.at[pl.ds(s * RPT + k * CHUNK, CHUNK)], zbuf_v)
    pltpu.sync_copy(
        zbuf_v, out_hbm.at[pl.ds(c * N_PAD + s * RPT + k * CHUNK, CHUNK)])


_deg_call = functools.partial(
    pl.kernel,
    out_type=jax.ShapeDtypeStruct((NC * N_PAD, DEGW), jnp.float32),
    mesh=_sc_mesh(),
    compiler_params=_SC_PARAMS,
    scratch_types=[
        pltpu.VMEM_SHARED((N_PAD, DEGW), jnp.float32),
        pltpu.VMEM((NCHUNK, CHUNK), jnp.int32),
        pltpu.VMEM((CHUNK, DEGW), jnp.float32),
        pltpu.VMEM((CHUNK, DEGW), jnp.float32),
    ],
)(_deg_body)


def _make_prop(d, nchunk0, nchunk1):
  nmax = max(nchunk0, nchunk1)

  def body(h_hbm, srcs0_hbm, dsts0_hbm, srcs1_hbm, dsts1_hbm, zeros_hbm,
           out_hbm, acc, srcs_v, dst0_v, dst1_v, rows0_v, rows1_v,
           semg0, semg1, semd0, semd1):
    c = lax.axis_index("c")
    s = lax.axis_index("s")
    rows = (rows0_v, rows1_v)
    dstb = (dst0_v, dst1_v)
    semg = (semg0, semg1)
    semd = (semd0, semd1)
    pltpu.sync_copy(zeros_hbm, rows0_v)
    for k in range(RPT // CHUNK):
      pltpu.sync_copy(rows0_v, acc.at[pl.ds(s * RPT + k * CHUNK, CHUNK)])

    def pipeline(sref, dref, nchunk):
      pltpu.sync_copy(sref.at[s], srcs_v.at[pl.ds(0, nchunk)])
      plsc.subcore_barrier()

      def gather(j, b):
        pltpu.async_copy(h_hbm.at[srcs_v.at[j]], rows[b], semg[b])
        pltpu.async_copy(dref.at[s].at[j], dstb[b], semd[b])

      def wait(b):
        pltpu.make_async_copy(
            h_hbm.at[pl.ds(0, CHUNK)], rows[b], semg[b]).wait()
        pltpu.make_async_copy(dref.at[s].at[0], dstb[b], semd[b]).wait()

      def scatter(b):
        pltpu.sync_copy(rows[b], acc.at[dstb[b]], add=True)

      gather(0, 0)
      gather(1, 1)

      def loop(j2, carry):
        j = 2 * j2
        wait(0)
        scatter(0)
        gather(j, 0)
        wait(1)
        scatter(1)
        gather(j + 1, 1)
        return carry

      lax.fori_loop(1, nchunk // 2, loop, 0, unroll=False)
      wait(0)
      scatter(0)
      wait(1)
      scatter(1)

    @pl.when(c == 0)
    def _():
      pipeline(srcs0_hbm, dsts0_hbm, nchunk0)

    @pl.when(c == 1)
    def _():
      pipeline(srcs1_hbm, dsts1_hbm, nchunk1)

    plsc.subcore_barrier()
    for k in range(RPT // CHUNK):
      pltpu.sync_copy(acc.at[pl.ds(s * RPT + k * CHUNK, CHUNK)], rows0_v)
      pltpu.sync_copy(
          rows0_v, out_hbm.at[pl.ds(c * N_PAD + s * RPT + k * CHUNK, CHUNK)])

  return pl.kernel(
      body,
      out_type=jax.ShapeDtypeStruct((NC * N_PAD, d), jnp.float32),
      mesh=_sc_mesh(),
      compiler_params=_SC_PARAMS,
      scratch_types=[
          pltpu.VMEM_SHARED((N_PAD, d), jnp.float32),
          pltpu.VMEM((nmax, CHUNK), jnp.int32),
          pltpu.VMEM((CHUNK,), jnp.int32),
          pltpu.VMEM((CHUNK,), jnp.int32),
          pltpu.VMEM((CHUNK, d), jnp.float32),
          pltpu.VMEM((CHUNK, d), jnp.float32),
          pltpu.SemaphoreType.DMA,
          pltpu.SemaphoreType.DMA,
          pltpu.SemaphoreType.DMA,
          pltpu.SemaphoreType.DMA,
      ],
  )


NCHUNK0 = 120
NCHUNK1 = 40
_prop128 = _make_prop(HIDDEN, NCHUNK0, NCHUNK1)
_prop64 = _make_prop(CLASSES, NCHUNK0, NCHUNK1)


def _norm_of(deg_ref):
  deg = deg_ref[0, :, 0:1] + deg_ref[1, :, 0:1]
  return lax.rsqrt(jnp.maximum(deg, 1.0))


def _tc1_body(deg_ref, feat_ref, w_ref, o_ref):
  norm = _norm_of(deg_ref)
  h = jnp.dot(feat_ref[...], w_ref[...], preferred_element_type=jnp.float32)
  o_ref[...] = h * norm


def _tc2_body(deg_ref, parts_ref, b_ref, w_ref, o_ref):
  norm = _norm_of(deg_ref)
  x = (parts_ref[0] + parts_ref[1]) * norm + b_ref[...]
  x = jnp.where(x > 0.0, x, jnp.exp(x) - 1.0)
  o_ref[...] = jnp.dot(
      x, w_ref[...], preferred_element_type=jnp.float32) * norm


def _tc3_body(deg_ref, parts_ref, b_ref, o_ref):
  norm = _norm_of(deg_ref)
  o_ref[...] = (parts_ref[0] + parts_ref[1]) * norm + b_ref[...]


def _deg_spec():
  return pl.BlockSpec((2, TCB, DEGW), lambda i: (0, i, 0))


def _tc1(deg_parts, feats, w1t):
  grid = (N_PAD // TCB,)
  return pl.pallas_call(
      _tc1_body,
      grid=grid,
      in_specs=[
          _deg_spec(),
          pl.BlockSpec((TCB, D_IN), lambda i: (i, 0)),
          pl.BlockSpec((D_IN, HIDDEN), lambda i: (0, 0)),
      ],
      out_specs=pl.BlockSpec((TCB, HIDDEN), lambda i: (i, 0)),
      out_shape=jax.ShapeDtypeStruct((N_PAD, HIDDEN), jnp.float32),
  )(deg_parts, feats, w1t)


def _tc2(deg_parts, parts1, b1r, w2t):
  grid = (N_PAD // TCB,)
  return pl.pallas_call(
      _tc2_body,
      grid=grid,
      in_specs=[
          _deg_spec(),
          pl.BlockSpec((2, TCB, HIDDEN), lambda i: (0, i, 0)),
          pl.BlockSpec((1, HIDDEN), lambda i: (0, 0)),
          pl.BlockSpec((HIDDEN, CLASSES), lambda i: (0, 0)),
      ],
      out_specs=pl.BlockSpec((TCB, CLASSES), lambda i: (i, 0)),
      out_shape=jax.ShapeDtypeStruct((N_PAD, CLASSES), jnp.float32),
  )(deg_parts, parts1, b1r, w2t)


def _tc3(deg_parts, parts2, b2r):
  grid = (N_PAD // TCB,)
  return pl.pallas_call(
      _tc3_body,
      grid=grid,
      in_specs=[
          _deg_spec(),
          pl.BlockSpec((2, TCB, CLASSES), lambda i: (0, i, 0)),
          pl.BlockSpec((1, CLASSES), lambda i: (0, 0)),
      ],
      out_specs=pl.BlockSpec((TCB, CLASSES), lambda i: (i, 0)),
      out_shape=jax.ShapeDtypeStruct((N_PAD, CLASSES), jnp.float32),
  )(deg_parts, parts2, b2r)


def kernel(features, edge_index, order_attn, W1, b1, W2, b2):
  del order_attn
  src = edge_index[0]
  dst = edge_index[1]
  pad = jnp.full((E_PAD - E,), N, dtype=jnp.int32)
  src_f = jnp.concatenate([src, pad])
  dst_f = jnp.concatenate([dst, pad])
  dsts = dst_f.reshape(NW, NCHUNK, CHUNK)
  e0 = NS * NCHUNK0 * CHUNK
  srcs0 = src_f[:e0].reshape(NS, NCHUNK0, CHUNK)
  dsts0 = dst_f[:e0].reshape(NS, NCHUNK0, CHUNK)
  srcs1 = src_f[e0:].reshape(NS, NCHUNK1, CHUNK)
  dsts1 = dst_f[e0:].reshape(NS, NCHUNK1, CHUNK)
  feats = jnp.pad(features, ((0, N_PAD - N), (0, 0)))

  onesw = jnp.ones((CHUNK, DEGW), jnp.float32)
  zw = jnp.zeros((CHUNK, DEGW), jnp.float32)
  z128 = jnp.zeros((CHUNK, HIDDEN), jnp.float32)
  z64 = jnp.zeros((CHUNK, CLASSES), jnp.float32)

  deg_parts = _deg_call(dsts, onesw, zw).reshape(NC, N_PAD, DEGW)

  h1 = _tc1(deg_parts, feats, W1.T)
  parts1 = _prop128(h1, srcs0, dsts0, srcs1, dsts1, z128).reshape(
      NC, N_PAD, HIDDEN)
  h2 = _tc2(deg_parts, parts1, b1.reshape(1, HIDDEN), W2.T)
  parts2 = _prop64(h2, srcs0, dsts0, srcs1, dsts1, z64).reshape(
      NC, N_PAD, CLASSES)
  out = _tc3(deg_parts, parts2, b2.reshape(1, CLASSES))
  return out[:N]


# speedup vs baseline: 2.0878x; 2.0878x over previous
import functools

import jax
import jax.numpy as jnp
from jax import lax
from jax.experimental import pallas as pl
from jax.experimental.pallas import tpu as pltpu
from jax.experimental.pallas import tpu_sc as plsc

N = 10000
E = 320000
D_IN = 128
HIDDEN = 128
CLASSES = 64

NC = 2
NS = 16
NW = NC * NS
CHUNK = 128
NCHUNK = 80
E_PAD = NW * NCHUNK * CHUNK
N_PAD = 10240
RPT = N_PAD // NS
TCB = 1024
DEGW = 16


_SC_PARAMS = pltpu.CompilerParams(use_tc_tiling_on_sc=False)


def _sc_mesh():
  return plsc.VectorSubcoreMesh(
      core_axis_name="c", subcore_axis_name="s", num_cores=NC,
      num_subcores=NS)


def _deg_body(dsts_hbm, ones_hbm, zeros_hbm, out_hbm, acc, dsts_v, ones_v,
              zbuf_v):
  c = lax.axis_index("c")
  s = lax.axis_index("s")
  wid = s * NC + c
  pltpu.sync_copy(zeros_hbm, zbuf_v)
  pltpu.sync_copy(ones_hbm, ones_v)
  pltpu.sync_copy(dsts_hbm.at[wid], dsts_v)
  for k in range(RPT // CHUNK):
    pltpu.sync_copy(zbuf_v, acc.at[pl.ds(s * RPT + k * CHUNK, CHUNK)])
  plsc.subcore_barrier()

  def body(j, carry):
    pltpu.sync_copy(ones_v, acc.at[dsts_v.at[j]], add=True)
    return carry

  lax.fori_loop(0, NCHUNK, body, 0)
  plsc.subcore_barrier()
  for k in range(RPT // CHUNK):
    pltpu.sync_copy(acc.at[pl.ds(s * RPT + k * CHUNK, CHUNK)], zbuf_v)
    pltpu.sync_copy(
        zbuf_v, out_hbm.at[pl.ds(c * N_PAD + s * RPT + k * CHUNK, CHUNK)])


_deg_call = functools.partial(
    pl.kernel,
    out_type=jax.ShapeDtypeStruct((NC * N_PAD, DEGW), jnp.float32),
    mesh=_sc_mesh(),
    compiler_params=_SC_PARAMS,
    scratch_types=[
        pltpu.VMEM_SHARED((N_PAD, DEGW), jnp.float32),
        pltpu.VMEM((NCHUNK, CHUNK), jnp.int32),
        pltpu.VMEM((CHUNK, DEGW), jnp.float32),
        pltpu.VMEM((CHUNK, DEGW), jnp.float32),
    ],
)(_deg_body)


def _make_prop(d, nchunk0, nchunk1):
  nmax = max(nchunk0, nchunk1)

  def body(h_hbm, srcs0_hbm, dsts0_hbm, srcs1_hbm, dsts1_hbm, zeros_hbm,
           out_hbm, acc, srcs_v, dst0_v, dst1_v, rows0_v, rows1_v,
           semg0, semg1, semd0, semd1):
    c = lax.axis_index("c")
    s = lax.axis_index("s")
    rows = (rows0_v, rows1_v)
    dstb = (dst0_v, dst1_v)
    semg = (semg0, semg1)
    semd = (semd0, semd1)
    pltpu.sync_copy(zeros_hbm, rows0_v)
    for k in range(RPT // CHUNK):
      pltpu.sync_copy(rows0_v, acc.at[pl.ds(s * RPT + k * CHUNK, CHUNK)])

    def pipeline(sref, dref, nchunk):
      pltpu.sync_copy(sref.at[s], srcs_v.at[pl.ds(0, nchunk)])
      plsc.subcore_barrier()

      def gather(j, b):
        pltpu.async_copy(h_hbm.at[srcs_v.at[j]], rows[b], semg[b])
        pltpu.async_copy(dref.at[s].at[j], dstb[b], semd[b])

      def wait(b):
        pltpu.make_async_copy(
            h_hbm.at[pl.ds(0, CHUNK)], rows[b], semg[b]).wait()
        pltpu.make_async_copy(dref.at[s].at[0], dstb[b], semd[b]).wait()

      def scatter(b):
        pltpu.sync_copy(rows[b], acc.at[dstb[b]], add=True)

      gather(0, 0)
      gather(1, 1)

      def loop(j2, carry):
        j = 2 * j2
        wait(0)
        scatter(0)
        gather(j, 0)
        wait(1)
        scatter(1)
        gather(j + 1, 1)
        return carry

      lax.fori_loop(1, nchunk // 2, loop, 0, unroll=False)
      wait(0)
      scatter(0)
      wait(1)
      scatter(1)

    @pl.when(c == 0)
    def _():
      pipeline(srcs0_hbm, dsts0_hbm, nchunk0)

    @pl.when(c == 1)
    def _():
      pipeline(srcs1_hbm, dsts1_hbm, nchunk1)

    plsc.subcore_barrier()
    for k in range(RPT // CHUNK):
      pltpu.sync_copy(acc.at[pl.ds(s * RPT + k * CHUNK, CHUNK)], rows0_v)
      pltpu.sync_copy(
          rows0_v, out_hbm.at[pl.ds(c * N_PAD + s * RPT + k * CHUNK, CHUNK)])

  return pl.kernel(
      body,
      out_type=jax.ShapeDtypeStruct((NC * N_PAD, d), jnp.float32),
      mesh=_sc_mesh(),
      compiler_params=_SC_PARAMS,
      scratch_types=[
          pltpu.VMEM_SHARED((N_PAD, d), jnp.float32),
          pltpu.VMEM((nmax, CHUNK), jnp.int32),
          pltpu.VMEM((CHUNK,), jnp.int32),
          pltpu.VMEM((CHUNK,), jnp.int32),
          pltpu.VMEM((CHUNK, d), jnp.float32),
          pltpu.VMEM((CHUNK, d), jnp.float32),
          pltpu.SemaphoreType.DMA,
          pltpu.SemaphoreType.DMA,
          pltpu.SemaphoreType.DMA,
          pltpu.SemaphoreType.DMA,
      ],
  )


NCHUNK0 = 120
NCHUNK1 = 40
_prop128 = _make_prop(HIDDEN, NCHUNK0, NCHUNK1)
_prop64 = _make_prop(CLASSES, NCHUNK0, NCHUNK1)

NCHUNK_T = E_PAD // (NS * CHUNK)


def _make_sprop(w):
  def body(ha_hbm, hb_hbm, srcs_hbm, dsts_hbm, zeros_hbm, out_hbm,
           h_sh, acc, srcs_v, dst0_v, dst1_v, rows0_v, rows1_v,
           semg0, semg1, semd0, semd1):
    c = lax.axis_index("c")
    s = lax.axis_index("s")
    rows = (rows0_v, rows1_v)
    dstb = (dst0_v, dst1_v)
    semg = (semg0, semg1)
    semd = (semd0, semd1)
    pltpu.sync_copy(zeros_hbm, rows0_v)
    pltpu.sync_copy(srcs_hbm.at[s], srcs_v)

    @pl.when(c == 0)
    def _():
      pltpu.sync_copy(ha_hbm.at[pl.ds(s * RPT, RPT)],
                      h_sh.at[pl.ds(s * RPT, RPT)])

    @pl.when(c == 1)
    def _():
      pltpu.sync_copy(hb_hbm.at[pl.ds(s * RPT, RPT)],
                      h_sh.at[pl.ds(s * RPT, RPT)])

    for k in range(RPT // CHUNK):
      pltpu.sync_copy(rows0_v, acc.at[pl.ds(s * RPT + k * CHUNK, CHUNK)])
    plsc.subcore_barrier()

    def gather(j, b):
      pltpu.async_copy(h_sh.at[srcs_v.at[j]], rows[b], semg[b])
      pltpu.async_copy(dsts_hbm.at[s].at[j], dstb[b], semd[b])

    def wait(b):
      pltpu.make_async_copy(
          h_sh.at[pl.ds(0, CHUNK)], rows[b], semg[b]).wait()
      pltpu.make_async_copy(dsts_hbm.at[s].at[0], dstb[b], semd[b]).wait()

    def scatter(b):
      pltpu.sync_copy(rows[b], acc.at[dstb[b]], add=True)

    gather(0, 0)
    gather(1, 1)

    def loop(j2, carry):
      j = 2 * j2
      wait(0)
      scatter(0)
      gather(j, 0)
      wait(1)
      scatter(1)
      gather(j + 1, 1)
      return carry

    lax.fori_loop(1, NCHUNK_T // 2, loop, 0, unroll=False)
    wait(0)
    scatter(0)
    wait(1)
    scatter(1)
    plsc.subcore_barrier()
    for k in range(RPT // CHUNK):
      pltpu.sync_copy(acc.at[pl.ds(s * RPT + k * CHUNK, CHUNK)], rows0_v)
      pltpu.sync_copy(
          rows0_v, out_hbm.at[pl.ds(c * N_PAD + s * RPT + k * CHUNK, CHUNK)])

  return pl.kernel(
      body,
      out_type=jax.ShapeDtypeStruct((NC * N_PAD, w), jnp.float32),
      mesh=_sc_mesh(),
      compiler_params=_SC_PARAMS,
      scratch_types=[
          pltpu.VMEM_SHARED((N_PAD, w), jnp.float32),
          pltpu.VMEM_SHARED((N_PAD, w), jnp.float32),
          pltpu.VMEM((NCHUNK_T, CHUNK), jnp.int32),
          pltpu.VMEM((CHUNK,), jnp.int32),
          pltpu.VMEM((CHUNK,), jnp.int32),
          pltpu.VMEM((CHUNK, w), jnp.float32),
          pltpu.VMEM((CHUNK, w), jnp.float32),
          pltpu.SemaphoreType.DMA,
          pltpu.SemaphoreType.DMA,
          pltpu.SemaphoreType.DMA,
          pltpu.SemaphoreType.DMA,
      ],
  )


_sprop64 = _make_sprop(HIDDEN // 2)
_sprop32 = _make_sprop(CLASSES // 2)


def _norm_of(deg_ref):
  deg = deg_ref[0, :, 0:1] + deg_ref[1, :, 0:1]
  return lax.rsqrt(jnp.maximum(deg, 1.0))


def _tc1_body(deg_ref, feat_ref, w_ref, oa_ref, ob_ref):
  norm = _norm_of(deg_ref)
  h = jnp.dot(feat_ref[...], w_ref[...], preferred_element_type=jnp.float32)
  h = h * norm
  oa_ref[...] = h[:, :HIDDEN // 2]
  ob_ref[...] = h[:, HIDDEN // 2:]


def _tc2_body(deg_ref, parts_ref, b_ref, w_ref, oa_ref, ob_ref):
  norm = _norm_of(deg_ref)
  agg = jnp.concatenate([parts_ref[0], parts_ref[1]], axis=1)
  x = agg * norm + b_ref[...]
  x = jnp.where(x > 0.0, x, jnp.exp(x) - 1.0)
  y = jnp.dot(x, w_ref[...], preferred_element_type=jnp.float32) * norm
  oa_ref[...] = y[:, :CLASSES // 2]
  ob_ref[...] = y[:, CLASSES // 2:]


def _tc3_body(deg_ref, parts_ref, b_ref, o_ref):
  norm = _norm_of(deg_ref)
  agg = jnp.concatenate([parts_ref[0], parts_ref[1]], axis=1)
  o_ref[...] = agg * norm + b_ref[...]


def _deg_spec():
  return pl.BlockSpec((2, TCB, DEGW), lambda i: (0, i, 0))


def _tc1(deg_parts, feats, w1t):
  grid = (N_PAD // TCB,)
  return pl.pallas_call(
      _tc1_body,
      grid=grid,
      in_specs=[
          _deg_spec(),
          pl.BlockSpec((TCB, D_IN), lambda i: (i, 0)),
          pl.BlockSpec((D_IN, HIDDEN), lambda i: (0, 0)),
      ],
      out_specs=[
          pl.BlockSpec((TCB, HIDDEN // 2), lambda i: (i, 0)),
          pl.BlockSpec((TCB, HIDDEN // 2), lambda i: (i, 0)),
      ],
      out_shape=[
          jax.ShapeDtypeStruct((N_PAD, HIDDEN // 2), jnp.float32),
          jax.ShapeDtypeStruct((N_PAD, HIDDEN // 2), jnp.float32),
      ],
  )(deg_parts, feats, w1t)


def _tc2(deg_parts, parts1, b1r, w2t):
  grid = (N_PAD // TCB,)
  return pl.pallas_call(
      _tc2_body,
      grid=grid,
      in_specs=[
          _deg_spec(),
          pl.BlockSpec((2, TCB, HIDDEN // 2), lambda i: (0, i, 0)),
          pl.BlockSpec((1, HIDDEN), lambda i: (0, 0)),
          pl.BlockSpec((HIDDEN, CLASSES), lambda i: (0, 0)),
      ],
      out_specs=[
          pl.BlockSpec((TCB, CLASSES // 2), lambda i: (i, 0)),
          pl.BlockSpec((TCB, CLASSES // 2), lambda i: (i, 0)),
      ],
      out_shape=[
          jax.ShapeDtypeStruct((N_PAD, CLASSES // 2), jnp.float32),
          jax.ShapeDtypeStruct((N_PAD, CLASSES // 2), jnp.float32),
      ],
  )(deg_parts, parts1, b1r, w2t)


def _tc3(deg_parts, parts2, b2r):
  grid = (N_PAD // TCB,)
  return pl.pallas_call(
      _tc3_body,
      grid=grid,
      in_specs=[
          _deg_spec(),
          pl.BlockSpec((2, TCB, CLASSES // 2), lambda i: (0, i, 0)),
          pl.BlockSpec((1, CLASSES), lambda i: (0, 0)),
      ],
      out_specs=pl.BlockSpec((TCB, CLASSES), lambda i: (i, 0)),
      out_shape=jax.ShapeDtypeStruct((N_PAD, CLASSES), jnp.float32),
  )(deg_parts, parts2, b2r)


def kernel(features, edge_index, order_attn, W1, b1, W2, b2):
  del order_attn
  src = edge_index[0]
  dst = edge_index[1]
  pad = jnp.full((E_PAD - E,), N, dtype=jnp.int32)
  src_f = jnp.concatenate([src, pad])
  dst_f = jnp.concatenate([dst, pad])
  dsts = dst_f.reshape(NW, NCHUNK, CHUNK)
  srcs_t = src_f.reshape(NS, NCHUNK_T, CHUNK)
  dsts_t = dst_f.reshape(NS, NCHUNK_T, CHUNK)
  feats = jnp.pad(features, ((0, N_PAD - N), (0, 0)))

  onesw = jnp.ones((CHUNK, DEGW), jnp.float32)
  zw = jnp.zeros((CHUNK, DEGW), jnp.float32)
  z64 = jnp.zeros((CHUNK, HIDDEN // 2), jnp.float32)
  z32 = jnp.zeros((CHUNK, CLASSES // 2), jnp.float32)

  deg_parts = _deg_call(dsts, onesw, zw).reshape(NC, N_PAD, DEGW)

  h1a, h1b = _tc1(deg_parts, feats, W1.T)
  parts1 = _sprop64(h1a, h1b, srcs_t, dsts_t, z64).reshape(
      NC, N_PAD, HIDDEN // 2)
  h2a, h2b = _tc2(deg_parts, parts1, b1.reshape(1, HIDDEN), W2.T)
  parts2 = _sprop32(h2a, h2b, srcs_t, dsts_t, z32).reshape(
      NC, N_PAD, CLASSES // 2)
  out = _tc3(deg_parts, parts2, b2.reshape(1, CLASSES))
  return out[:N]
